# baseline (speedup 1.0000x reference)
import functools

import jax
import jax.numpy as jnp
from jax import lax
from jax.experimental import pallas as pl
from jax.experimental.pallas import tpu as pltpu
from jax.experimental.pallas import tpu_sc as plsc

B, J, D = 16384, 26, 128
BB = 1024

_mesh = plsc.VectorSubcoreMesh(core_axis_name="c", subcore_axis_name="s")


@functools.partial(
    pl.kernel,
    out_type=jax.ShapeDtypeStruct((128,), jnp.float32),
    mesh=_mesh,
    scratch_types=[
        pltpu.VMEM((128,), jnp.int32),
        pltpu.VMEM((128,), jnp.float32),
        pltpu.SemaphoreType.DMA,
    ],
)
def _sc_scales(idx_hbm, lw_hbm, sig_hbm, idx_v, w_v, sem):
    cid = lax.axis_index("c")
    sid = lax.axis_index("s")

    @pl.when(jnp.logical_and(cid == 0, sid == 0))
    def _():
        pltpu.sync_copy(idx_hbm, idx_v)
        pltpu.async_copy(lw_hbm.at[idx_v], w_v, sem).wait()
        for t in range(128 // 16):
            wv = w_v[pl.ds(16 * t, 16)]
            w_v[pl.ds(16 * t, 16)] = 1.0 / (1.0 + jnp.exp(-wv))
        pltpu.sync_copy(w_v, sig_hbm)


PROBE_P = 6
_RPW = B // 32
_PCH = 256


@functools.partial(
    pl.kernel,
    out_type=jax.ShapeDtypeStruct((PROBE_P * B * D,), jnp.float32),
    mesh=_mesh,
    scratch_types=[
        pltpu.VMEM((_PCH * D,), jnp.float32),
        pltpu.VMEM((_PCH * D,), jnp.float32),
        pltpu.SemaphoreType.DMA,
        pltpu.SemaphoreType.DMA,
    ],
)
def _sc_probe_copy(feat_hbm, out_hbm, buf0, buf1, sem0, sem1):
    cid = lax.axis_index("c")
    sid = lax.axis_index("s")
    wid = sid * 2 + cid
    nch = _RPW // _PCH

    def plane(p, c):
        def chunk(t, c2):
            base = (p * B + wid * _RPW + t * _PCH) * D
            pltpu.async_copy(feat_hbm.at[pl.ds(base, _PCH * D)], buf0,
                             sem0).wait()
            pltpu.async_copy(buf0, out_hbm.at[pl.ds(base, _PCH * D)],
                             sem1).wait()
            return c2
        return lax.fori_loop(0, nch, chunk, c)

    lax.fori_loop(0, PROBE_P, plane, 0)


def _tc_body(sig_ref, feat_ref, out_ref):
    for j in range(J):
        out_ref[j] = feat_ref[j] * sig_ref[j]


_tc_mul = pl.pallas_call(
    _tc_body,
    grid=(B // BB,),
    in_specs=[
        pl.BlockSpec(memory_space=pltpu.SMEM),
        pl.BlockSpec((J, BB, D), lambda i: (0, i, 0)),
    ],
    out_specs=pl.BlockSpec((J, BB, D), lambda i: (0, i, 0)),
    out_shape=jax.ShapeDtypeStruct((J, B, D), jnp.float32),
)


def kernel(idx, feat, layerweight):
    idxp = jnp.zeros((128,), jnp.int32).at[:J].set(
        idx.reshape(J).astype(jnp.int32))
    lwp = jnp.zeros((128,), jnp.float32).at[:100].set(layerweight)
    sig = _sc_scales(idxp, lwp)
    ft = jnp.transpose(feat, (1, 0, 2))
    dummy = _sc_probe_copy(ft.reshape(-1))
    out_t = _tc_mul(sig, ft)
    eps = dummy[0] - dummy[0]
    out_t = out_t.at[0, 0, 0].add(eps)
    return jnp.transpose(out_t, (1, 0, 2))


# speedup vs baseline: 1.0651x; 1.0651x over previous
import functools

import jax
import jax.numpy as jnp
from jax import lax
from jax.experimental import pallas as pl
from jax.experimental.pallas import tpu as pltpu
from jax.experimental.pallas import tpu_sc as plsc

B, J, D = 16384, 26, 128
BB = 1024

_mesh = plsc.VectorSubcoreMesh(core_axis_name="c", subcore_axis_name="s")


@functools.partial(
    pl.kernel,
    out_type=jax.ShapeDtypeStruct((128,), jnp.float32),
    mesh=_mesh,
    scratch_types=[
        pltpu.VMEM((128,), jnp.int32),
        pltpu.VMEM((128,), jnp.float32),
        pltpu.SemaphoreType.DMA,
    ],
)
def _sc_scales(idx_hbm, lw_hbm, sig_hbm, idx_v, w_v, sem):
    cid = lax.axis_index("c")
    sid = lax.axis_index("s")

    @pl.when(jnp.logical_and(cid == 0, sid == 0))
    def _():
        pltpu.sync_copy(idx_hbm, idx_v)
        pltpu.async_copy(lw_hbm.at[idx_v], w_v, sem).wait()
        for t in range(128 // 16):
            wv = w_v[pl.ds(16 * t, 16)]
            w_v[pl.ds(16 * t, 16)] = 1.0 / (1.0 + jnp.exp(-wv))
        pltpu.sync_copy(w_v, sig_hbm)


PROBE_P = 6
_RPW = B // 32
_PCH = 256


@functools.partial(
    pl.kernel,
    out_type=jax.ShapeDtypeStruct((PROBE_P * B * D,), jnp.float32),
    mesh=_mesh,
    scratch_types=[
        pltpu.VMEM((_PCH * D,), jnp.float32),
        pltpu.VMEM((_PCH * D,), jnp.float32),
        pltpu.SemaphoreType.DMA,
        pltpu.SemaphoreType.DMA,
    ],
)
def _sc_probe_copy(feat_hbm, out_hbm, buf0, buf1, sem0, sem1):
    cid = lax.axis_index("c")
    sid = lax.axis_index("s")
    wid = sid * 2 + cid
    nch = _RPW // _PCH

    def plane(p, c):
        def chunk(t, c2):
            base = (p * B + wid * _RPW + t * _PCH) * D
            pltpu.async_copy(feat_hbm.at[pl.ds(base, _PCH * D)], buf0,
                             sem0).wait()
            pltpu.async_copy(buf0, out_hbm.at[pl.ds(base, _PCH * D)],
                             sem1).wait()
            return c2
        return lax.fori_loop(0, nch, chunk, c)

    lax.fori_loop(0, PROBE_P, plane, 0)


def _t_body(idx_ref, lw_ref, out_ref):
    k = lax.broadcasted_iota(jnp.int32, (128, 128), 0)
    idxb = jnp.broadcast_to(idx_ref[...], (128, 128))
    lwb = jnp.broadcast_to(lw_ref[...], (128, 128))
    w = jnp.sum(jnp.where(k == idxb, lwb, 0.0), axis=0, keepdims=True)
    out_ref[...] = 1.0 / (1.0 + jnp.exp(-w))


_t_scales = pl.pallas_call(
    _t_body,
    out_shape=jax.ShapeDtypeStruct((1, 128), jnp.float32),
)


def _tc_body(sig_ref, feat_ref, out_ref):
    for j in range(J):
        out_ref[j] = feat_ref[j] * sig_ref[j]


_tc_mul = pl.pallas_call(
    _tc_body,
    grid=(B // BB,),
    in_specs=[
        pl.BlockSpec(memory_space=pltpu.SMEM),
        pl.BlockSpec((J, BB, D), lambda i: (0, i, 0)),
    ],
    out_specs=pl.BlockSpec((J, BB, D), lambda i: (0, i, 0)),
    out_shape=jax.ShapeDtypeStruct((J, B, D), jnp.float32),
)


def kernel(idx, feat, layerweight):
    idxp = jnp.zeros((128,), jnp.int32).at[:J].set(
        idx.reshape(J).astype(jnp.int32))
    lwp = jnp.zeros((128,), jnp.float32).at[:100].set(layerweight)
    sig = _t_scales(idxp.reshape(1, 128), lwp.reshape(128, 1))
    ft = jnp.transpose(feat, (1, 0, 2))
    dummy = _sc_probe_copy(ft.reshape(-1))
    out_t = _tc_mul(sig.reshape(128), ft)
    eps = dummy[0] - dummy[0]
    out_t = out_t.at[0, 0, 0].add(eps)
    return jnp.transpose(out_t, (1, 0, 2))


# speedup vs baseline: 1.3214x; 1.2406x over previous
import functools

import jax
import jax.numpy as jnp
from jax import lax
from jax.experimental import pallas as pl
from jax.experimental.pallas import tpu as pltpu
from jax.experimental.pallas import tpu_sc as plsc

B, J, D = 16384, 26, 128
BB = 1024

_mesh = plsc.VectorSubcoreMesh(core_axis_name="c", subcore_axis_name="s")


@functools.partial(
    pl.kernel,
    out_type=jax.ShapeDtypeStruct((32,), jnp.float32),
    mesh=_mesh,
    scratch_types=[
        pltpu.VMEM((32,), jnp.int32),
        pltpu.VMEM((32,), jnp.float32),
        pltpu.SemaphoreType.DMA,
    ],
)
def _sc_scales(idx_hbm, lw_hbm, sig_hbm, idx_v, w_v, sem):
    cid = lax.axis_index("c")
    sid = lax.axis_index("s")

    @pl.when(jnp.logical_and(cid == 0, sid == 0))
    def _():
        pltpu.sync_copy(idx_hbm, idx_v)
        pltpu.async_copy(lw_hbm.at[idx_v], w_v, sem).wait()
        for t in range(2):
            wv = w_v[pl.ds(16 * t, 16)]
            w_v[pl.ds(16 * t, 16)] = 1.0 / (1.0 + jnp.exp(-wv))
        pltpu.sync_copy(w_v, sig_hbm)


def _tc_body(sig_ref, feat_ref, out_ref):
    for j in range(J):
        out_ref[j] = feat_ref[j] * sig_ref[j]


_tc_mul = pl.pallas_call(
    _tc_body,
    grid=(B // BB,),
    in_specs=[
        pl.BlockSpec(memory_space=pltpu.SMEM),
        pl.BlockSpec((J, BB, D), lambda i: (0, i, 0)),
    ],
    out_specs=pl.BlockSpec((J, BB, D), lambda i: (0, i, 0)),
    out_shape=jax.ShapeDtypeStruct((J, B, D), jnp.float32),
)


def kernel(idx, feat, layerweight):
    idxp = jnp.zeros((32,), jnp.int32).at[:J].set(
        idx.reshape(J).astype(jnp.int32))
    sig = _sc_scales(idxp, layerweight)
    out_t = _tc_mul(sig, jnp.transpose(feat, (1, 0, 2)))
    return jnp.transpose(out_t, (1, 0, 2))
